# baseline (speedup 1.0000x reference)
import functools

import jax
import jax.numpy as jnp
from jax import lax
from jax.experimental import pallas as pl
from jax.experimental.pallas import tpu as pltpu
from jax.experimental.pallas import tpu_sc as plsc

_BATCH = 16384
_NUM_CORES = 1
_NUM_SUBCORES = 8
_NUM_WORKERS = _NUM_CORES * _NUM_SUBCORES
_PER_WORKER = _BATCH // _NUM_WORKERS
_CHUNK = 128
_NUM_CHUNKS = _PER_WORKER // _CHUNK

_mesh = plsc.VectorSubcoreMesh(core_axis_name="c", subcore_axis_name="s", num_cores=1, num_subcores=8)


@functools.partial(
    pl.kernel,
    mesh=_mesh,
    out_type=jax.ShapeDtypeStruct((_NUM_WORKERS, _NUM_CHUNKS, _CHUNK), jnp.float32),
    scratch_types=[
        pltpu.VMEM((_NUM_CHUNKS, _CHUNK), jnp.int32),
        pltpu.VMEM((_NUM_CHUNKS, _CHUNK), jnp.float32),
    ]
    + [pltpu.SemaphoreType.DMA] * _NUM_CHUNKS
    + [pltpu.SemaphoreType.DMA],
)
def _gather_kernel(w_hbm, idx_hbm, out_hbm, idx_v, vals_v, *sems):
    gsems, osem = sems[:_NUM_CHUNKS], sems[_NUM_CHUNKS]
    wid = lax.axis_index("s") * _NUM_CORES + lax.axis_index("c")
    pltpu.sync_copy(idx_hbm.at[wid], idx_v)
    table = w_hbm.at[0]
    gathers = [
        pltpu.async_copy(table.at[idx_v.at[j]], vals_v.at[j], gsems[j])
        for j in range(_NUM_CHUNKS)
    ]
    stores = []
    for j in range(_NUM_CHUNKS):
        gathers[j].wait()
        stores.append(
            pltpu.async_copy(vals_v.at[j], out_hbm.at[wid].at[j], osem)
        )
    for s in stores:
        s.wait()


def kernel(cur_step, W):
    idx = cur_step.astype(jnp.int32).reshape(_NUM_WORKERS, _NUM_CHUNKS, _CHUNK)
    out = _gather_kernel(W, idx)
    return out.reshape(_BATCH, 1)


# speedup vs baseline: 1.0606x; 1.0606x over previous
import functools

import jax
import jax.numpy as jnp
from jax import lax
from jax.experimental import pallas as pl
from jax.experimental.pallas import tpu as pltpu
from jax.experimental.pallas import tpu_sc as plsc

_BATCH = 16384
_NUM_CORES = 1
_NUM_SUBCORES = 16
_NUM_WORKERS = _NUM_CORES * _NUM_SUBCORES
_PER_WORKER = _BATCH // _NUM_WORKERS
_CHUNK = 128
_NUM_CHUNKS = _PER_WORKER // _CHUNK

_mesh = plsc.VectorSubcoreMesh(core_axis_name="c", subcore_axis_name="s", num_cores=1)


@functools.partial(
    pl.kernel,
    mesh=_mesh,
    out_type=jax.ShapeDtypeStruct((_NUM_WORKERS, _NUM_CHUNKS, _CHUNK), jnp.float32),
    scratch_types=[
        pltpu.VMEM((_NUM_CHUNKS, _CHUNK), jnp.int32),
        pltpu.VMEM((_NUM_CHUNKS, _CHUNK), jnp.float32),
    ]
    + [pltpu.SemaphoreType.DMA] * _NUM_CHUNKS
    + [pltpu.SemaphoreType.DMA],
)
def _gather_kernel(w_hbm, idx_hbm, out_hbm, idx_v, vals_v, *sems):
    gsems, osem = sems[:_NUM_CHUNKS], sems[_NUM_CHUNKS]
    wid = lax.axis_index("s") * _NUM_CORES + lax.axis_index("c")
    pltpu.sync_copy(idx_hbm.at[wid], idx_v)
    table = w_hbm.at[0]
    gathers = [
        pltpu.async_copy(table.at[idx_v.at[j]], vals_v.at[j], gsems[j])
        for j in range(_NUM_CHUNKS)
    ]
    stores = []
    for j in range(_NUM_CHUNKS):
        gathers[j].wait()
        stores.append(
            pltpu.async_copy(vals_v.at[j], out_hbm.at[wid].at[j], osem)
        )
    for s in stores:
        s.wait()


def kernel(cur_step, W):
    idx = cur_step.astype(jnp.int32).reshape(_NUM_WORKERS, _NUM_CHUNKS, _CHUNK)
    out = _gather_kernel(W, idx)
    return out.reshape(_BATCH, 1)


# speedup vs baseline: 1.0678x; 1.0067x over previous
import functools

import jax
import jax.numpy as jnp
from jax import lax
from jax.experimental import pallas as pl
from jax.experimental.pallas import tpu as pltpu
from jax.experimental.pallas import tpu_sc as plsc

_BATCH = 16384
_NUM_CORES = 1
_NUM_SUBCORES = 16
_NUM_WORKERS = _NUM_CORES * _NUM_SUBCORES
_PER_WORKER = _BATCH // _NUM_WORKERS
_CHUNK = 128
_NUM_CHUNKS = _PER_WORKER // _CHUNK

_mesh = plsc.VectorSubcoreMesh(core_axis_name="c", subcore_axis_name="s", num_cores=1)


@functools.partial(
    pl.kernel,
    mesh=_mesh,
    out_type=jax.ShapeDtypeStruct((_NUM_WORKERS, _NUM_CHUNKS, _CHUNK), jnp.float32),
    scratch_types=[
        pltpu.VMEM((_NUM_CHUNKS, _CHUNK), jnp.int32),
        pltpu.VMEM((_NUM_CHUNKS, _CHUNK), jnp.float32),
    ]
    + [pltpu.SemaphoreType.DMA] * _NUM_CHUNKS,
)
def _gather_kernel(w_hbm, idx_hbm, out_hbm, idx_v, vals_v, *gsems):
    wid = lax.axis_index("s") * _NUM_CORES + lax.axis_index("c")
    pltpu.sync_copy(idx_hbm.at[wid], idx_v)
    table = w_hbm.at[0]
    gathers = [
        pltpu.async_copy(table.at[idx_v.at[j]], vals_v.at[j], gsems[j])
        for j in range(_NUM_CHUNKS)
    ]
    stores = []
    for j in range(_NUM_CHUNKS):
        gathers[j].wait()
        stores.append(
            pltpu.async_copy(vals_v.at[j], out_hbm.at[wid].at[j], gsems[j])
        )
    for s in stores:
        s.wait()


def kernel(cur_step, W):
    idx = cur_step.astype(jnp.int32).reshape(_NUM_WORKERS, _NUM_CHUNKS, _CHUNK)
    out = _gather_kernel(W, idx)
    return out.reshape(_BATCH, 1)
